# baseline (speedup 1.0000x reference)
import functools

import jax
import jax.numpy as jnp
from jax import lax
from jax.experimental import pallas as pl
from jax.experimental.pallas import tpu as pltpu
from jax.experimental.pallas import tpu_sc as plsc


_N, _L, _D, _H, _E, _M, _P = 4, 512, 768, 12, 64, 3, 1024
_NW = 32
_EPW = (_N * _H * _E) // _NW
_EC = 32


def _sc_gather_sum(att_hbm, ridx_hbm, out_hbm, idx_v, rows_v, out_v, sem):
    wid = lax.axis_index("s") * 2 + lax.axis_index("c")
    for c in range(_EPW // _EC):
        base_e = wid * _EPW + c * _EC
        pltpu.sync_copy(ridx_hbm.at[pl.ds(base_e * _M, _EC * _M)], idx_v)
        pltpu.async_copy(att_hbm.at[idx_v], rows_v, sem).wait()

        def body_e(e, _):
            def body_g(g, _):
                s = pl.ds(g * 16, 16)
                out_v[e, s] = (rows_v[_M * e, s] + rows_v[_M * e + 1, s]
                               + rows_v[_M * e + 2, s])
                return 0
            return lax.fori_loop(0, _L // 16, body_g, 0)
        lax.fori_loop(0, _EC, body_e, 0)
        pltpu.sync_copy(out_v, out_hbm.at[pl.ds(base_e, _EC)])


_sc_gather = functools.partial(
    pl.kernel,
    out_type=jax.ShapeDtypeStruct((_N * _H * _E, _L), jnp.float32),
    mesh=plsc.VectorSubcoreMesh(core_axis_name="c", subcore_axis_name="s"),
    scratch_types=[
        pltpu.VMEM((_EC * _M,), jnp.int32),
        pltpu.VMEM((_EC * _M, _L), jnp.float32),
        pltpu.VMEM((_EC, _L), jnp.float32),
        pltpu.SemaphoreType.DMA,
    ],
)(_sc_gather_sum)


def _hrt_kernel(pos_ref, hidx_ref, tidx_ref, seq_ref, esum_ref,
                hs_ref, ts_ref, rs_ref):
    seq = seq_ref[0]
    seq16 = seq.astype(jnp.bfloat16)
    pos = pos_ref[0, 0, :]
    hidx = hidx_ref[0, 0, :]
    tidx = tidx_ref[0, 0, :]

    l_iota = jax.lax.broadcasted_iota(jnp.int32, (_E * _M, _L), 1)
    poh = (pos[:, None] == l_iota).astype(jnp.bfloat16)

    mention = jnp.dot(poh, seq16, preferred_element_type=jnp.float32)
    me = mention.reshape(_E, _M, _D)
    mmax = jnp.max(me, axis=1)
    e_emb = mmax + jnp.log(jnp.sum(jnp.exp(me - mmax[:, None, :]), axis=1))

    e_iota = jax.lax.broadcasted_iota(jnp.int32, (_P, _E), 1)
    oh_h = (hidx[:, None] == e_iota).astype(jnp.bfloat16)
    oh_t = (tidx[:, None] == e_iota).astype(jnp.bfloat16)

    e_att_cols = [
        (esum_ref[0, hh] * (1.0 / _M)).astype(jnp.bfloat16)
        for hh in range(_H)
    ]
    e_att_all = jnp.concatenate(e_att_cols, axis=1)

    hc = 4
    acc = jnp.zeros((_P, _L), jnp.float32)
    for c in range(_H // hc):
        ec = e_att_all[:, c * hc * _L:(c + 1) * hc * _L]
        h_att = jnp.dot(oh_h, ec, preferred_element_type=jnp.float32)
        t_att = jnp.dot(oh_t, ec, preferred_element_type=jnp.float32)
        prod = h_att * t_att
        for k in range(hc):
            acc = acc + prod[:, k * _L:(k + 1) * _L]

    ht_att = acc * (1.0 / _H)
    ht_att = ht_att / (jnp.sum(ht_att, axis=1, keepdims=True) + 1e-5)

    rs_ref[0] = jnp.dot(ht_att.astype(jnp.bfloat16), seq16,
                        preferred_element_type=jnp.float32)
    e_emb16 = e_emb.astype(jnp.bfloat16)
    hs_ref[0] = jnp.dot(oh_h, e_emb16, preferred_element_type=jnp.float32)
    ts_ref[0] = jnp.dot(oh_t, e_emb16, preferred_element_type=jnp.float32)


def kernel(sequence_output, attention, entity_pos, hts):
    n, L, d = sequence_output.shape
    h = attention.shape[1]
    E, M = entity_pos.shape[1], entity_pos.shape[2]
    P = hts.shape[1]
    assert (n, L, d, h, E, M, P) == (_N, _L, _D, _H, _E, _M, _P)

    pos1 = (entity_pos[:, :, :, 0] + 1).astype(jnp.int32)
    base = (jnp.arange(n * h, dtype=jnp.int32).reshape(n, h, 1, 1)) * L
    ridx = (base + pos1[:, None, :, :]).reshape(n * h * E * M)

    esum = _sc_gather(attention.reshape(n * h * L, L), ridx)
    esum = esum.reshape(n, h, E, L)

    pos = pos1.reshape(n, 1, E * M)
    hidx = hts[:, :, 0].reshape(n, 1, P).astype(jnp.int32)
    tidx = hts[:, :, 1].reshape(n, 1, P).astype(jnp.int32)

    out_shape = [jax.ShapeDtypeStruct((n, P, d), jnp.float32)] * 3
    hs, ts, rs = pl.pallas_call(
        _hrt_kernel,
        grid=(n,),
        in_specs=[
            pl.BlockSpec((1, 1, E * M), lambda i: (i, 0, 0)),
            pl.BlockSpec((1, 1, P), lambda i: (i, 0, 0)),
            pl.BlockSpec((1, 1, P), lambda i: (i, 0, 0)),
            pl.BlockSpec((1, L, d), lambda i: (i, 0, 0)),
            pl.BlockSpec((1, h, E, L), lambda i: (i, 0, 0, 0)),
        ],
        out_specs=[
            pl.BlockSpec((1, P, d), lambda i: (i, 0, 0)),
            pl.BlockSpec((1, P, d), lambda i: (i, 0, 0)),
            pl.BlockSpec((1, P, d), lambda i: (i, 0, 0)),
        ],
        out_shape=out_shape,
        compiler_params=pltpu.CompilerParams(
            vmem_limit_bytes=100 * 1024 * 1024),
    )(pos, hidx, tidx, sequence_output, esum)

    return hs.reshape(-1, d), ts.reshape(-1, d), rs.reshape(-1, d)


# speedup vs baseline: 1.0417x; 1.0417x over previous
import functools

import jax
import jax.numpy as jnp
from jax import lax
from jax.experimental import pallas as pl
from jax.experimental.pallas import tpu as pltpu
from jax.experimental.pallas import tpu_sc as plsc


_N, _L, _D, _H, _E, _M, _P = 4, 512, 768, 12, 64, 3, 1024
_NW = 32
_EPW = (_N * _H * _E) // _NW
_EC = 32


def _sc_gather_sum(att_hbm, ridx_hbm, out_hbm, idx_v, rows_v, out_v, sem):
    wid = lax.axis_index("s") * 2 + lax.axis_index("c")
    for c in range(_EPW // _EC):
        base_e = wid * _EPW + c * _EC
        pltpu.sync_copy(ridx_hbm.at[pl.ds(base_e * _M, _EC * _M)], idx_v)
        pltpu.async_copy(att_hbm.at[idx_v], rows_v, sem).wait()

        def body_e(e, _):
            for g in range(_L // 16):
                s = pl.ds(g * 16, 16)
                out_v[e, s] = (rows_v[_M * e, s] + rows_v[_M * e + 1, s]
                               + rows_v[_M * e + 2, s])
            return 0
        lax.fori_loop(0, _EC, body_e, 0)
        pltpu.sync_copy(out_v, out_hbm.at[pl.ds(base_e, _EC)])


_sc_gather = functools.partial(
    pl.kernel,
    out_type=jax.ShapeDtypeStruct((_N * _H * _E, _L), jnp.float32),
    mesh=plsc.VectorSubcoreMesh(core_axis_name="c", subcore_axis_name="s"),
    scratch_types=[
        pltpu.VMEM((_EC * _M,), jnp.int32),
        pltpu.VMEM((_EC * _M, _L), jnp.float32),
        pltpu.VMEM((_EC, _L), jnp.float32),
        pltpu.SemaphoreType.DMA,
    ],
)(_sc_gather_sum)


def _entity_onehots(hidx, tidx):
    e_iota = jax.lax.broadcasted_iota(jnp.int32, (_P, _E), 1)
    oh_h = (hidx[:, None] == e_iota).astype(jnp.bfloat16)
    oh_t = (tidx[:, None] == e_iota).astype(jnp.bfloat16)
    return oh_h, oh_t


def _tc_emb_kernel(pos_ref, hidx_ref, tidx_ref, seq_ref, hs_ref, ts_ref):
    seq16 = seq_ref[0].astype(jnp.bfloat16)
    pos = pos_ref[0, 0, :]

    l_iota = jax.lax.broadcasted_iota(jnp.int32, (_E * _M, _L), 1)
    poh = (pos[:, None] == l_iota).astype(jnp.bfloat16)

    mention = jnp.dot(poh, seq16, preferred_element_type=jnp.float32)
    me = mention.reshape(_E, _M, _D)
    mmax = jnp.max(me, axis=1)
    e_emb = mmax + jnp.log(jnp.sum(jnp.exp(me - mmax[:, None, :]), axis=1))
    e_emb16 = e_emb.astype(jnp.bfloat16)

    oh_h, oh_t = _entity_onehots(hidx_ref[0, 0, :], tidx_ref[0, 0, :])
    hs_ref[0] = jnp.dot(oh_h, e_emb16, preferred_element_type=jnp.float32)
    ts_ref[0] = jnp.dot(oh_t, e_emb16, preferred_element_type=jnp.float32)


def _tc_pair_kernel(hidx_ref, tidx_ref, seq_ref, esum_ref, rs_ref):
    oh_h, oh_t = _entity_onehots(hidx_ref[0, 0, :], tidx_ref[0, 0, :])

    e_att_cols = [
        (esum_ref[0, hh] * (1.0 / _M)).astype(jnp.bfloat16)
        for hh in range(_H)
    ]
    e_att_all = jnp.concatenate(e_att_cols, axis=1)

    hc = 4
    acc = jnp.zeros((_P, _L), jnp.float32)
    for c in range(_H // hc):
        ec = e_att_all[:, c * hc * _L:(c + 1) * hc * _L]
        h_att = jnp.dot(oh_h, ec, preferred_element_type=jnp.float32)
        t_att = jnp.dot(oh_t, ec, preferred_element_type=jnp.float32)
        prod = h_att * t_att
        for k in range(hc):
            acc = acc + prod[:, k * _L:(k + 1) * _L]

    ht_att = acc * (1.0 / _H)
    ht_att = ht_att / (jnp.sum(ht_att, axis=1, keepdims=True) + 1e-5)

    rs_ref[0] = jnp.dot(ht_att.astype(jnp.bfloat16),
                        seq_ref[0].astype(jnp.bfloat16),
                        preferred_element_type=jnp.float32)


def kernel(sequence_output, attention, entity_pos, hts):
    n, L, d = sequence_output.shape
    h = attention.shape[1]
    E, M = entity_pos.shape[1], entity_pos.shape[2]
    P = hts.shape[1]
    assert (n, L, d, h, E, M, P) == (_N, _L, _D, _H, _E, _M, _P)

    pos1 = (entity_pos[:, :, :, 0] + 1).astype(jnp.int32)
    base = (jnp.arange(n * h, dtype=jnp.int32).reshape(n, h, 1, 1)) * L
    ridx = (base + pos1[:, None, :, :]).reshape(n * h * E * M)

    esum = _sc_gather(attention.reshape(n * h * L, L), ridx)
    esum = esum.reshape(n, h, E, L)

    pos = pos1.reshape(n, 1, E * M)
    hidx = hts[:, :, 0].reshape(n, 1, P).astype(jnp.int32)
    tidx = hts[:, :, 1].reshape(n, 1, P).astype(jnp.int32)

    idx_spec = pl.BlockSpec((1, 1, P), lambda i: (i, 0, 0))
    seq_spec = pl.BlockSpec((1, L, d), lambda i: (i, 0, 0))
    out_spec = pl.BlockSpec((1, P, d), lambda i: (i, 0, 0))

    hs, ts = pl.pallas_call(
        _tc_emb_kernel,
        grid=(n,),
        in_specs=[
            pl.BlockSpec((1, 1, E * M), lambda i: (i, 0, 0)),
            idx_spec, idx_spec, seq_spec,
        ],
        out_specs=[out_spec, out_spec],
        out_shape=[jax.ShapeDtypeStruct((n, P, d), jnp.float32)] * 2,
    )(pos, hidx, tidx, sequence_output)

    rs = pl.pallas_call(
        _tc_pair_kernel,
        grid=(n,),
        in_specs=[
            idx_spec, idx_spec, seq_spec,
            pl.BlockSpec((1, h, E, L), lambda i: (i, 0, 0, 0)),
        ],
        out_specs=out_spec,
        out_shape=jax.ShapeDtypeStruct((n, P, d), jnp.float32),
    )(hidx, tidx, sequence_output, esum)

    return hs.reshape(-1, d), ts.reshape(-1, d), rs.reshape(-1, d)


# speedup vs baseline: 1.0815x; 1.0382x over previous
import functools

import jax
import jax.numpy as jnp
from jax import lax
from jax.experimental import pallas as pl
from jax.experimental.pallas import tpu as pltpu
from jax.experimental.pallas import tpu_sc as plsc


_N, _L, _D, _H, _E, _M, _P = 4, 512, 768, 12, 64, 3, 1024
_NW = 32
_EPW = (_N * _H * _E) // _NW
_EC = 32


def _sc_gather_sum(att_hbm, ridx_hbm, out_hbm, idx_v, rows_v, out_v, sem0,
                   sem1):
    wid = lax.axis_index("s") * 2 + lax.axis_index("c")
    nchunk = _EPW // _EC
    sems = (sem0, sem1)
    pltpu.sync_copy(ridx_hbm.at[pl.ds(wid * _EPW * _M, _EPW * _M)], idx_v)

    def start(c):
        return pltpu.async_copy(
            att_hbm.at[idx_v.at[pl.ds(c * _EC * _M, _EC * _M)]],
            rows_v.at[c % 2], sems[c % 2])

    copies = {0: start(0)}
    for c in range(nchunk):
        if c + 1 < nchunk:
            copies[c + 1] = start(c + 1)
        copies[c].wait()
        buf = c % 2

        def body_e(e, _):
            for g in range(_L // 16):
                s = pl.ds(g * 16, 16)
                out_v[e, s] = (rows_v[buf, _M * e, s]
                               + rows_v[buf, _M * e + 1, s]
                               + rows_v[buf, _M * e + 2, s])
            return 0
        lax.fori_loop(0, _EC, body_e, 0)
        pltpu.sync_copy(out_v, out_hbm.at[pl.ds(wid * _EPW + c * _EC, _EC)])


_sc_gather = functools.partial(
    pl.kernel,
    out_type=jax.ShapeDtypeStruct((_N * _H * _E, _L), jnp.float32),
    mesh=plsc.VectorSubcoreMesh(core_axis_name="c", subcore_axis_name="s"),
    scratch_types=[
        pltpu.VMEM((_EPW * _M,), jnp.int32),
        pltpu.VMEM((2, _EC * _M, _L), jnp.float32),
        pltpu.VMEM((_EC, _L), jnp.float32),
        pltpu.SemaphoreType.DMA,
        pltpu.SemaphoreType.DMA,
    ],
)(_sc_gather_sum)


def _entity_onehots(hidx, tidx):
    e_iota = jax.lax.broadcasted_iota(jnp.int32, (_P, _E), 1)
    oh_h = (hidx[:, None] == e_iota).astype(jnp.bfloat16)
    oh_t = (tidx[:, None] == e_iota).astype(jnp.bfloat16)
    return oh_h, oh_t


def _tc_emb_kernel(pos_ref, hidx_ref, tidx_ref, seq_ref, hs_ref, ts_ref):
    seq16 = seq_ref[0].astype(jnp.bfloat16)
    pos = pos_ref[0, 0, :]

    l_iota = jax.lax.broadcasted_iota(jnp.int32, (_E * _M, _L), 1)
    poh = (pos[:, None] == l_iota).astype(jnp.bfloat16)

    mention = jnp.dot(poh, seq16, preferred_element_type=jnp.float32)
    me = mention.reshape(_E, _M, _D)
    mmax = jnp.max(me, axis=1)
    e_emb = mmax + jnp.log(jnp.sum(jnp.exp(me - mmax[:, None, :]), axis=1))
    e_emb16 = e_emb.astype(jnp.bfloat16)

    oh_h, oh_t = _entity_onehots(hidx_ref[0, 0, :], tidx_ref[0, 0, :])
    hs_ref[0] = jnp.dot(oh_h, e_emb16, preferred_element_type=jnp.float32)
    ts_ref[0] = jnp.dot(oh_t, e_emb16, preferred_element_type=jnp.float32)


def _tc_pair_kernel(hidx_ref, tidx_ref, seq_ref, esum_ref, rs_ref):
    oh_h, oh_t = _entity_onehots(hidx_ref[0, 0, :], tidx_ref[0, 0, :])

    e_att_cols = [
        (esum_ref[0, hh] * (1.0 / _M)).astype(jnp.bfloat16)
        for hh in range(_H)
    ]
    e_att_all = jnp.concatenate(e_att_cols, axis=1)

    hc = 4
    acc = jnp.zeros((_P, _L), jnp.float32)
    for c in range(_H // hc):
        ec = e_att_all[:, c * hc * _L:(c + 1) * hc * _L]
        h_att = jnp.dot(oh_h, ec, preferred_element_type=jnp.float32)
        t_att = jnp.dot(oh_t, ec, preferred_element_type=jnp.float32)
        prod = h_att * t_att
        for k in range(hc):
            acc = acc + prod[:, k * _L:(k + 1) * _L]

    ht_att = acc * (1.0 / _H)
    ht_att = ht_att / (jnp.sum(ht_att, axis=1, keepdims=True) + 1e-5)

    rs_ref[0] = jnp.dot(ht_att.astype(jnp.bfloat16),
                        seq_ref[0].astype(jnp.bfloat16),
                        preferred_element_type=jnp.float32)


def kernel(sequence_output, attention, entity_pos, hts):
    n, L, d = sequence_output.shape
    h = attention.shape[1]
    E, M = entity_pos.shape[1], entity_pos.shape[2]
    P = hts.shape[1]
    assert (n, L, d, h, E, M, P) == (_N, _L, _D, _H, _E, _M, _P)

    pos1 = (entity_pos[:, :, :, 0] + 1).astype(jnp.int32)
    base = (jnp.arange(n * h, dtype=jnp.int32).reshape(n, h, 1, 1)) * L
    ridx = (base + pos1[:, None, :, :]).reshape(n * h * E * M)

    esum = _sc_gather(attention.reshape(n * h * L, L), ridx)
    esum = esum.reshape(n, h, E, L)

    pos = pos1.reshape(n, 1, E * M)
    hidx = hts[:, :, 0].reshape(n, 1, P).astype(jnp.int32)
    tidx = hts[:, :, 1].reshape(n, 1, P).astype(jnp.int32)

    idx_spec = pl.BlockSpec((1, 1, P), lambda i: (i, 0, 0))
    seq_spec = pl.BlockSpec((1, L, d), lambda i: (i, 0, 0))
    out_spec = pl.BlockSpec((1, P, d), lambda i: (i, 0, 0))

    hs, ts = pl.pallas_call(
        _tc_emb_kernel,
        grid=(n,),
        in_specs=[
            pl.BlockSpec((1, 1, E * M), lambda i: (i, 0, 0)),
            idx_spec, idx_spec, seq_spec,
        ],
        out_specs=[out_spec, out_spec],
        out_shape=[jax.ShapeDtypeStruct((n, P, d), jnp.float32)] * 2,
    )(pos, hidx, tidx, sequence_output)

    rs = pl.pallas_call(
        _tc_pair_kernel,
        grid=(n,),
        in_specs=[
            idx_spec, idx_spec, seq_spec,
            pl.BlockSpec((1, h, E, L), lambda i: (i, 0, 0, 0)),
        ],
        out_specs=out_spec,
        out_shape=jax.ShapeDtypeStruct((n, P, d), jnp.float32),
    )(hidx, tidx, sequence_output, esum)

    return hs.reshape(-1, d), ts.reshape(-1, d), rs.reshape(-1, d)


# speedup vs baseline: 1.6709x; 1.5451x over previous
import jax
import jax.numpy as jnp
from jax.experimental import pallas as pl
from jax.experimental.pallas import tpu as pltpu


_N, _L, _D, _H, _E, _M, _P = 4, 512, 768, 12, 64, 3, 1024


def _hrt_kernel(pos_ref, hidx_ref, tidx_ref, seq_ref, att_ref,
                hs_ref, ts_ref, rs_ref):
    seq = seq_ref[0]
    seq16 = seq.astype(jnp.bfloat16)
    pos = pos_ref[0, 0, :]
    hidx = hidx_ref[0, 0, :]
    tidx = tidx_ref[0, 0, :]

    l_iota = jax.lax.broadcasted_iota(jnp.int32, (_E * _M, _L), 1)
    poh = (pos[:, None] == l_iota).astype(jnp.bfloat16)

    mention = jnp.dot(poh, seq16, preferred_element_type=jnp.float32)
    me = mention.reshape(_E, _M, _D)
    mmax = jnp.max(me, axis=1)
    e_emb = mmax + jnp.log(jnp.sum(jnp.exp(me - mmax[:, None, :]), axis=1))

    w16 = (poh.reshape(_E, _M, _L).sum(axis=1) * (1.0 / _M))

    e_iota = jax.lax.broadcasted_iota(jnp.int32, (_P, _E), 1)
    oh_h = (hidx[:, None] == e_iota).astype(jnp.bfloat16)
    oh_t = (tidx[:, None] == e_iota).astype(jnp.bfloat16)

    e_att_cols = []
    for hh in range(_H):
        att_h = att_ref[0, hh].astype(jnp.bfloat16)
        e_att_cols.append(jnp.dot(w16, att_h,
                                  preferred_element_type=jnp.float32
                                  ).astype(jnp.bfloat16))
    e_att_all = jnp.concatenate(e_att_cols, axis=1)

    hc = 4
    acc = jnp.zeros((_P, _L), jnp.float32)
    for c in range(_H // hc):
        ec = e_att_all[:, c * hc * _L:(c + 1) * hc * _L]
        h_att = jnp.dot(oh_h, ec, preferred_element_type=jnp.float32)
        t_att = jnp.dot(oh_t, ec, preferred_element_type=jnp.float32)
        prod = h_att * t_att
        for k in range(hc):
            acc = acc + prod[:, k * _L:(k + 1) * _L]

    ht_att = acc * (1.0 / _H)
    ht_att = ht_att / (jnp.sum(ht_att, axis=1, keepdims=True) + 1e-5)

    rs_ref[0] = jnp.dot(ht_att.astype(jnp.bfloat16), seq16,
                        preferred_element_type=jnp.float32)
    e_emb16 = e_emb.astype(jnp.bfloat16)
    hs_ref[0] = jnp.dot(oh_h, e_emb16, preferred_element_type=jnp.float32)
    ts_ref[0] = jnp.dot(oh_t, e_emb16, preferred_element_type=jnp.float32)


def kernel(sequence_output, attention, entity_pos, hts):
    n, L, d = sequence_output.shape
    h = attention.shape[1]
    E, M = entity_pos.shape[1], entity_pos.shape[2]
    P = hts.shape[1]
    assert (n, L, d, h, E, M, P) == (_N, _L, _D, _H, _E, _M, _P)

    pos = (entity_pos[:, :, :, 0].reshape(n, 1, E * M) + 1).astype(jnp.int32)
    hidx = hts[:, :, 0].reshape(n, 1, P).astype(jnp.int32)
    tidx = hts[:, :, 1].reshape(n, 1, P).astype(jnp.int32)

    out_shape = [jax.ShapeDtypeStruct((n, P, d), jnp.float32)] * 3
    hs, ts, rs = pl.pallas_call(
        _hrt_kernel,
        grid=(n,),
        in_specs=[
            pl.BlockSpec((1, 1, E * M), lambda i: (i, 0, 0)),
            pl.BlockSpec((1, 1, P), lambda i: (i, 0, 0)),
            pl.BlockSpec((1, 1, P), lambda i: (i, 0, 0)),
            pl.BlockSpec((1, L, d), lambda i: (i, 0, 0)),
            pl.BlockSpec((1, h, L, L), lambda i: (i, 0, 0, 0)),
        ],
        out_specs=[
            pl.BlockSpec((1, P, d), lambda i: (i, 0, 0)),
            pl.BlockSpec((1, P, d), lambda i: (i, 0, 0)),
            pl.BlockSpec((1, P, d), lambda i: (i, 0, 0)),
        ],
        out_shape=out_shape,
        compiler_params=pltpu.CompilerParams(
            vmem_limit_bytes=100 * 1024 * 1024),
    )(pos, hidx, tidx, sequence_output, attention)

    return hs.reshape(-1, d), ts.reshape(-1, d), rs.reshape(-1, d)
